# baseline (speedup 1.0000x reference)
import functools

import jax
import jax.numpy as jnp
from jax import lax
from jax.experimental import pallas as pl
from jax.experimental.pallas import tpu as pltpu
from jax.experimental.pallas import tpu_sc as plsc

_CHUNK = 128


def kernel(labels, embedding_table, train):
    del train
    B = labels.shape[0]
    V, D = embedding_table.shape

    info = plsc.get_sparse_core_info()
    NC, NS = info.num_cores, info.num_subcores
    NW = NC * NS
    b_per_w = B // NW
    n_chunks = b_per_w // _CHUNK

    idx3 = labels.reshape(NW, n_chunks, _CHUNK)
    mesh = plsc.VectorSubcoreMesh(core_axis_name="c", subcore_axis_name="s")

    @functools.partial(
        pl.kernel,
        mesh=mesh,
        out_type=jax.ShapeDtypeStruct((B, D), jnp.float32),
        scratch_types=[
            pltpu.VMEM((n_chunks, _CHUNK), jnp.int32),
            pltpu.VMEM((b_per_w, D), jnp.float32),
            pltpu.SemaphoreType.DMA,
        ],
        compiler_params=pltpu.CompilerParams(use_tc_tiling_on_sc=False),
    )
    def emb(idx_hbm, table_hbm, out_hbm, idx_v, rows_v, sem):
        wid = lax.axis_index("s") * NC + lax.axis_index("c")
        base = wid * b_per_w
        pltpu.sync_copy(idx_hbm.at[wid], idx_v)
        copies = []
        for j in range(n_chunks):
            copies.append(
                pltpu.async_copy(
                    table_hbm.at[idx_v.at[j]],
                    rows_v.at[pl.ds(j * _CHUNK, _CHUNK)],
                    sem,
                )
            )
        for c in copies:
            c.wait()
        pltpu.sync_copy(rows_v, out_hbm.at[pl.ds(base, b_per_w)])

    return emb(idx3, embedding_table)


# speedup vs baseline: 1.6452x; 1.6452x over previous
import functools

import jax
import jax.numpy as jnp
from jax import lax
from jax.experimental import pallas as pl
from jax.experimental.pallas import tpu as pltpu
from jax.experimental.pallas import tpu_sc as plsc

_K = 16


def kernel(labels, embedding_table, train):
    del train
    B = labels.shape[0]
    V, D = embedding_table.shape

    info = plsc.get_sparse_core_info()
    NC, NS = info.num_cores, info.num_subcores
    NW = NC * NS
    b_per_w = B // NW
    n_groups = b_per_w // _K

    mesh = plsc.VectorSubcoreMesh(core_axis_name="c", subcore_axis_name="s")

    @functools.partial(
        pl.kernel,
        mesh=mesh,
        out_type=jax.ShapeDtypeStruct((B, D), jnp.float32),
        scratch_types=[
            pltpu.VMEM((b_per_w,), jnp.int32),
            pltpu.VMEM((b_per_w, D), jnp.float32),
            pltpu.SemaphoreType.DMA,
        ],
    )
    def emb(idx_hbm, table_hbm, out_hbm, idx_s, rows_v, sem):
        wid = lax.axis_index("s") * NC + lax.axis_index("c")
        base = wid * b_per_w
        pltpu.sync_copy(idx_hbm.at[pl.ds(base, b_per_w)], idx_s)

        def group(g):
            vec = idx_s[pl.ds(g * _K, _K)]
            for j in range(_K):
                i = g * _K + j
                r = vec[j]
                pltpu.async_copy(
                    table_hbm.at[pl.ds(r, 1)], rows_v.at[pl.ds(i, 1)], sem
                )
            pltpu.make_async_copy(
                table_hbm.at[pl.ds(0, _K)], rows_v.at[pl.ds(0, _K)], sem
            ).wait()

        pl.loop(0, n_groups)(group)
        pltpu.sync_copy(rows_v, out_hbm.at[pl.ds(base, b_per_w)])

    return emb(labels, embedding_table)


# speedup vs baseline: 3.0066x; 1.8275x over previous
import functools

import jax
import jax.numpy as jnp
from jax import lax
from jax.experimental import pallas as pl
from jax.experimental.pallas import tpu as pltpu
from jax.experimental.pallas import tpu_sc as plsc

_CHUNK = 128


def kernel(labels, embedding_table, train):
    del train
    B = labels.shape[0]
    V, D = embedding_table.shape

    info = plsc.get_sparse_core_info()
    NC, NS = info.num_cores, info.num_subcores
    d_per_c = D // NC
    b_per_s = B // NS
    n_chunks = b_per_s // _CHUNK

    mesh = plsc.VectorSubcoreMesh(core_axis_name="c", subcore_axis_name="s")

    @functools.partial(
        pl.kernel,
        mesh=mesh,
        out_type=jax.ShapeDtypeStruct((D, B), jnp.float32),
        scratch_types=[
            pltpu.VMEM((b_per_s,), jnp.int32),
            pltpu.VMEM((d_per_c * b_per_s,), jnp.float32),
            pltpu.VMEM_SHARED((V,), jnp.float32),
            pltpu.SemaphoreType.DMA,
            pltpu.SemaphoreType.DMA,
            pltpu.SemaphoreType.DMA,
        ],
    )
    def emb(idx_hbm, tab_hbm, out_hbm, idx_v, out_v, row0, row_sem,
            g_sem, w_sem):
        cid = lax.axis_index("c")
        sid = lax.axis_index("s")
        ibase = pl.multiple_of(sid * b_per_s, b_per_s)
        d0 = cid * d_per_c

        pltpu.sync_copy(idx_hbm.at[pl.ds(ibase, b_per_s)], idx_v)

        @pl.when(sid == 0)
        def _():
            pltpu.async_copy(tab_hbm.at[d0], row0, row_sem)

        def do_feature(k):
            @pl.when(sid == 0)
            def _():
                pltpu.make_async_copy(tab_hbm.at[d0], row0, row_sem).wait()

            plsc.subcore_barrier()

            obase = pl.multiple_of(k * b_per_s, b_per_s)
            copies = []
            for m in range(n_chunks):
                copies.append(
                    pltpu.async_copy(
                        row0.at[idx_v.at[pl.ds(m * _CHUNK, _CHUNK)]],
                        out_v.at[pl.ds(obase + m * _CHUNK, _CHUNK)],
                        g_sem,
                    )
                )
            for c in copies:
                c.wait()
            pltpu.async_copy(
                out_v.at[pl.ds(obase, b_per_s)],
                out_hbm.at[d0 + k, pl.ds(ibase, b_per_s)],
                w_sem,
            )
            plsc.subcore_barrier()

            @pl.when(jnp.logical_and(sid == 0, k + 1 < d_per_c))
            def _():
                pltpu.async_copy(tab_hbm.at[d0 + k + 1], row0, row_sem)

        pl.loop(0, d_per_c)(do_feature)

        pltpu.make_async_copy(
            tab_hbm.at[0, pl.ds(0, d_per_c * b_per_s)], out_v, w_sem
        ).wait()

    return emb(labels, embedding_table.T).T
